# baseline (speedup 1.0000x reference)
import functools

import jax
import jax.numpy as jnp
from jax import lax
from jax.experimental import pallas as pl
from jax.experimental.pallas import tpu as pltpu
from jax.experimental.pallas import tpu_sc as plsc

DIM = 128
NODES = 10000
NREL = 500
NRELP = 512
EDGES = 320000

NC = 2
NS = 16
NW = NC * NS
CHUNK = 56
NCHUNK = 180
EPW = NCHUNK * CHUNK
EDGES_PAD = NW * EPW
NPAD = 10240
RPT = NPAD // NS
WB = 40
LANES = 16
NBUF = 3
IR = 4
PERIOD = 12


def _rel_body(rel_ref, wrel_ref, brel_ref, gate_ref, outrel_ref):
    r = rel_ref[...]
    gate_ref[...] = 1.0 / (1.0 + jnp.exp(-r))
    acc = lax.dot_general(r[:NREL], wrel_ref[...], (((1,), (1,)), ((), ())),
                          preferred_element_type=jnp.float32)
    outrel_ref[...] = acc + brel_ref[...]


def _ent_body(ent_ref, p0_ref, p1_ref, ws_ref, wn_ref, bs_ref, bn_ref,
              out_ref):
    agg = p0_ref[...] + p1_ref[...]
    acc = lax.dot_general(ent_ref[...], ws_ref[...], (((1,), (1,)), ((), ())),
                          preferred_element_type=jnp.float32)
    acc = acc + lax.dot_general(agg, wn_ref[...], (((1,), (1,)), ((), ())),
                                preferred_element_type=jnp.float32)
    out_ref[...] = jnp.maximum(acc + bs_ref[...] + bn_ref[...], 0.0)


def _sc_body(idx_hbm, ent_hbm, gate_hbm, out_hbm,
             rows0, rows1, rows2, gat0, gat1, gat2,
             idxc0, idxc1, idxc2, idxc3,
             agg_sh,
             gsem0, gsem1, gsem2, ssem0, ssem1, ssem2,
             isem0, isem1, isem2, isem3):
    rows = [rows0, rows1, rows2]
    gat = [gat0, gat1, gat2]
    idxc = [idxc0, idxc1, idxc2, idxc3]
    gsem = [gsem0, gsem1, gsem2]
    ssem = [ssem0, ssem1, ssem2]
    isem = [isem0, isem1, isem2, isem3]

    c = lax.axis_index("c")
    s = lax.axis_index("s")
    wid = s * NC + c
    ebase = pl.multiple_of(wid * EPW, CHUNK)

    def idx_descs(j, r):
        goff = pl.multiple_of(3 * (ebase + j * CHUNK), 3 * CHUNK)
        return (
            pltpu.make_async_copy(idx_hbm.at[pl.ds(goff, 3 * CHUNK)],
                                  idxc[r], isem[r]),
        )

    def gather_descs(p, r):
        return (
            pltpu.make_async_copy(ent_hbm.at[idxc[r].at[pl.ds(0, CHUNK)]],
                                  rows[p], gsem[p]),
            pltpu.make_async_copy(gate_hbm.at[idxc[r].at[pl.ds(CHUNK, CHUNK)]],
                                  gat[p], gsem[p]),
        )

    def scatter_desc(p, r):
        return pltpu.make_async_copy(
            rows[p], agg_sh.at[idxc[r].at[pl.ds(2 * CHUNK, CHUNK)]], ssem[p])

    zero = jnp.zeros((LANES,), jnp.float32)

    @pl.loop(0, WB)
    def _zero_rows(i):
        for k in range(DIM // LANES):
            rows0[i, pl.ds(k * LANES, LANES)] = zero

    base = s * RPT
    for b in range(RPT // WB):
        pltpu.sync_copy(rows0.at[pl.ds(0, WB)],
                        agg_sh.at[pl.ds(base + b * WB, WB)])
    plsc.subcore_barrier()

    for d in idx_descs(0, 0):
        d.start()
    for d in idx_descs(1, 1):
        d.start()
    for d in idx_descs(2, 2):
        d.start()
    for d in idx_descs(0, 0):
        d.wait()
    for d in gather_descs(0, 0):
        d.start()
    for d in idx_descs(1, 1):
        d.wait()
    for d in gather_descs(1, 1):
        d.start()

    @pl.loop(0, NCHUNK, step=PERIOD)
    def _outer(j0):
        for q in range(PERIOD):
            j = j0 + q
            p = q % NBUF
            pg = (q + 2) % NBUF
            r = q % IR
            rg = (q + 2) % IR
            rn = (q + 3) % IR

            for d in gather_descs(p, r):
                d.wait()

            @plsc.parallel_loop(0, CHUNK, unroll=8)
            def _row(i):
                for k in range(DIM // LANES):
                    sl = pl.ds(k * LANES, LANES)
                    rows[p][i, sl] = rows[p][i, sl] * gat[p][i, sl]

            @pl.when(j > 0)
            def _wait_prev_scatter():
                scatter_desc(pg, rn).wait()

            @pl.when(j + 2 < NCHUNK)
            def _issue_gather():
                for d in idx_descs(j + 2, rg):
                    d.wait()
                for d in gather_descs(pg, rg):
                    d.start()

            @pl.when(j + 3 < NCHUNK)
            def _issue_idx():
                for d in idx_descs(j + 3, rn):
                    d.start()

            pltpu.async_copy(rows[p],
                             agg_sh.at[idxc[r].at[pl.ds(2 * CHUNK, CHUNK)]],
                             ssem[p], add=True)

    scatter_desc((NCHUNK - 1) % NBUF, (NCHUNK - 1) % IR).wait()
    plsc.subcore_barrier()

    for b in range(RPT // WB):
        sl = pl.ds(base + b * WB, WB)
        pltpu.sync_copy(agg_sh.at[sl], rows0.at[pl.ds(0, WB)])
        pltpu.sync_copy(rows0.at[pl.ds(0, WB)], out_hbm.at[c, sl])


_sc_edges = functools.partial(
    pl.kernel,
    out_type=jax.ShapeDtypeStruct((NC, NPAD, DIM), jnp.float32),
    mesh=plsc.VectorSubcoreMesh(core_axis_name="c", subcore_axis_name="s"),
    scratch_types=(
        [pltpu.VMEM((CHUNK, DIM), jnp.float32) for _ in range(6)]
        + [pltpu.VMEM((3 * CHUNK,), jnp.int32) for _ in range(4)]
        + [pltpu.VMEM_SHARED((NPAD, DIM), jnp.float32)]
        + [pltpu.SemaphoreType.DMA for _ in range(10)]
    ),
)(_sc_body)


@jax.jit
def kernel(ent, rel, edge_index, edge_type, W_self, b_self, W_nei, b_nei,
           W_rel, b_rel):
    npad = EDGES_PAD - EDGES
    src = jnp.concatenate(
        [edge_index[0].astype(jnp.int32), jnp.zeros((npad,), jnp.int32)])
    dst = jnp.concatenate(
        [edge_index[1].astype(jnp.int32), jnp.full((npad,), NODES, jnp.int32)])
    typ = jnp.concatenate(
        [edge_type.astype(jnp.int32), jnp.zeros((npad,), jnp.int32)])
    idx = jnp.stack(
        [src.reshape(NW * NCHUNK, CHUNK),
         typ.reshape(NW * NCHUNK, CHUNK),
         dst.reshape(NW * NCHUNK, CHUNK)],
        axis=1).reshape(-1)

    gate, out_rel = pl.pallas_call(
        _rel_body,
        out_shape=[
            jax.ShapeDtypeStruct((NRELP, DIM), jnp.float32),
            jax.ShapeDtypeStruct((NREL, DIM), jnp.float32),
        ],
    )(jnp.concatenate([rel, jnp.zeros((NRELP - NREL, DIM), jnp.float32)]),
      W_rel, b_rel.reshape(1, DIM))

    parts = _sc_edges(idx, ent, gate)

    rows_blk = 2000
    grid = NODES // rows_blk
    out_ent = pl.pallas_call(
        _ent_body,
        grid=(grid,),
        in_specs=[
            pl.BlockSpec((rows_blk, DIM), lambda i: (i, 0)),
            pl.BlockSpec((rows_blk, DIM), lambda i: (i, 0)),
            pl.BlockSpec((rows_blk, DIM), lambda i: (i, 0)),
            pl.BlockSpec((DIM, DIM), lambda i: (0, 0)),
            pl.BlockSpec((DIM, DIM), lambda i: (0, 0)),
            pl.BlockSpec((1, DIM), lambda i: (0, 0)),
            pl.BlockSpec((1, DIM), lambda i: (0, 0)),
        ],
        out_specs=pl.BlockSpec((rows_blk, DIM), lambda i: (i, 0)),
        out_shape=jax.ShapeDtypeStruct((NODES, DIM), jnp.float32),
    )(ent, parts[0, :NODES], parts[1, :NODES], W_self, W_nei,
      b_self.reshape(1, DIM), b_nei.reshape(1, DIM))

    return (out_ent, out_rel)


# speedup vs baseline: 1.0086x; 1.0086x over previous
import functools

import jax
import jax.numpy as jnp
from jax import lax
from jax.experimental import pallas as pl
from jax.experimental.pallas import tpu as pltpu
from jax.experimental.pallas import tpu_sc as plsc

DIM = 128
NODES = 10000
NREL = 500
NRELP = 512
EDGES = 320000

NC = 2
NS = 16
NW = NC * NS
CHUNK = 56
NCHUNK = 180
EPW = NCHUNK * CHUNK
EDGES_PAD = NW * EPW
NPAD = 10240
RPT = NPAD // NS
WB = 40
LANES = 16
NBUF = 3
IR = 4
PERIOD = 12


def _rel_body(rel_ref, wrel_ref, brel_ref, gate_ref, outrel_ref):
    r = rel_ref[...]
    gate_ref[...] = 1.0 / (1.0 + jnp.exp(-r))
    acc = lax.dot_general(r[:NREL], wrel_ref[...], (((1,), (1,)), ((), ())),
                          preferred_element_type=jnp.float32)
    outrel_ref[...] = acc + brel_ref[...]


def _ent_body(ent_ref, p0_ref, p1_ref, ws_ref, wn_ref, bs_ref, bn_ref,
              out_ref):
    agg = p0_ref[...] + p1_ref[...]
    acc = lax.dot_general(ent_ref[...], ws_ref[...], (((1,), (1,)), ((), ())),
                          preferred_element_type=jnp.float32)
    acc = acc + lax.dot_general(agg, wn_ref[...], (((1,), (1,)), ((), ())),
                                preferred_element_type=jnp.float32)
    out_ref[...] = jnp.maximum(acc + bs_ref[...] + bn_ref[...], 0.0)


def _sc_body(idx_hbm, ent_hbm, gate_hbm, out_hbm,
             rows0, rows1, rows2, gat0, gat1, gat2,
             idxc0, idxc1, idxc2, idxc3,
             agg_sh,
             gsem0, gsem1, gsem2, ssem0, ssem1, ssem2,
             isem0, isem1, isem2, isem3):
    rows = [rows0, rows1, rows2]
    gat = [gat0, gat1, gat2]
    idxc = [idxc0, idxc1, idxc2, idxc3]
    gsem = [gsem0, gsem1, gsem2]
    ssem = [ssem0, ssem1, ssem2]
    isem = [isem0, isem1, isem2, isem3]

    c = lax.axis_index("c")
    s = lax.axis_index("s")
    wid = s * NC + c
    ebase = pl.multiple_of(wid * EPW, CHUNK)

    def idx_descs(j, r):
        goff = pl.multiple_of(3 * (ebase + j * CHUNK), 3 * CHUNK)
        return (
            pltpu.make_async_copy(idx_hbm.at[pl.ds(goff, 3 * CHUNK)],
                                  idxc[r], isem[r]),
        )

    def gather_descs(p, r):
        return (
            pltpu.make_async_copy(ent_hbm.at[idxc[r].at[pl.ds(0, CHUNK)]],
                                  rows[p], gsem[p]),
            pltpu.make_async_copy(gate_hbm.at[idxc[r].at[pl.ds(CHUNK, CHUNK)]],
                                  gat[p], gsem[p]),
        )

    def scatter_desc(p, r):
        return pltpu.make_async_copy(
            rows[p], agg_sh.at[idxc[r].at[pl.ds(2 * CHUNK, CHUNK)]], ssem[p])

    zero = jnp.zeros((LANES,), jnp.float32)

    @pl.loop(0, WB)
    def _zero_rows(i):
        for k in range(DIM // LANES):
            rows0[i, pl.ds(k * LANES, LANES)] = zero

    base = s * RPT
    for b in range(RPT // WB):
        pltpu.sync_copy(rows0.at[pl.ds(0, WB)],
                        agg_sh.at[pl.ds(base + b * WB, WB)])
    plsc.subcore_barrier()

    for d in idx_descs(0, 0):
        d.start()
    for d in idx_descs(1, 1):
        d.start()
    for d in idx_descs(2, 2):
        d.start()
    for d in idx_descs(0, 0):
        d.wait()
    for d in gather_descs(0, 0):
        d.start()
    for d in idx_descs(1, 1):
        d.wait()
    for d in gather_descs(1, 1):
        d.start()

    @pl.loop(0, NCHUNK, step=PERIOD)
    def _outer(j0):
        for q in range(PERIOD):
            j = j0 + q
            p = q % NBUF
            pg = (q + 2) % NBUF
            r = q % IR
            rg = (q + 2) % IR
            rn = (q + 3) % IR

            for d in gather_descs(p, r):
                d.wait()

            @plsc.parallel_loop(0, CHUNK, unroll=4)
            def _row(i):
                for k in range(DIM // LANES):
                    sl = pl.ds(k * LANES, LANES)
                    rows[p][i, sl] = rows[p][i, sl] * gat[p][i, sl]

            @pl.when(j > 0)
            def _wait_prev_scatter():
                scatter_desc(pg, rn).wait()

            @pl.when(j + 2 < NCHUNK)
            def _issue_gather():
                for d in idx_descs(j + 2, rg):
                    d.wait()
                for d in gather_descs(pg, rg):
                    d.start()

            @pl.when(j + 3 < NCHUNK)
            def _issue_idx():
                for d in idx_descs(j + 3, rn):
                    d.start()

            pltpu.async_copy(rows[p],
                             agg_sh.at[idxc[r].at[pl.ds(2 * CHUNK, CHUNK)]],
                             ssem[p], add=True)

    scatter_desc((NCHUNK - 1) % NBUF, (NCHUNK - 1) % IR).wait()
    plsc.subcore_barrier()

    for b in range(RPT // WB):
        sl = pl.ds(base + b * WB, WB)
        pltpu.sync_copy(agg_sh.at[sl], rows0.at[pl.ds(0, WB)])
        pltpu.sync_copy(rows0.at[pl.ds(0, WB)], out_hbm.at[c, sl])


_sc_edges = functools.partial(
    pl.kernel,
    out_type=jax.ShapeDtypeStruct((NC, NPAD, DIM), jnp.float32),
    mesh=plsc.VectorSubcoreMesh(core_axis_name="c", subcore_axis_name="s"),
    scratch_types=(
        [pltpu.VMEM((CHUNK, DIM), jnp.float32) for _ in range(6)]
        + [pltpu.VMEM((3 * CHUNK,), jnp.int32) for _ in range(4)]
        + [pltpu.VMEM_SHARED((NPAD, DIM), jnp.float32)]
        + [pltpu.SemaphoreType.DMA for _ in range(10)]
    ),
)(_sc_body)


@jax.jit
def kernel(ent, rel, edge_index, edge_type, W_self, b_self, W_nei, b_nei,
           W_rel, b_rel):
    npad = EDGES_PAD - EDGES
    src = jnp.concatenate(
        [edge_index[0].astype(jnp.int32), jnp.zeros((npad,), jnp.int32)])
    dst = jnp.concatenate(
        [edge_index[1].astype(jnp.int32), jnp.full((npad,), NODES, jnp.int32)])
    typ = jnp.concatenate(
        [edge_type.astype(jnp.int32), jnp.zeros((npad,), jnp.int32)])
    idx = jnp.stack(
        [src.reshape(NW * NCHUNK, CHUNK),
         typ.reshape(NW * NCHUNK, CHUNK),
         dst.reshape(NW * NCHUNK, CHUNK)],
        axis=1).reshape(-1)

    gate, out_rel = pl.pallas_call(
        _rel_body,
        out_shape=[
            jax.ShapeDtypeStruct((NRELP, DIM), jnp.float32),
            jax.ShapeDtypeStruct((NREL, DIM), jnp.float32),
        ],
    )(jnp.concatenate([rel, jnp.zeros((NRELP - NREL, DIM), jnp.float32)]),
      W_rel, b_rel.reshape(1, DIM))

    parts = _sc_edges(idx, ent, gate)

    rows_blk = 2000
    grid = NODES // rows_blk
    out_ent = pl.pallas_call(
        _ent_body,
        grid=(grid,),
        in_specs=[
            pl.BlockSpec((rows_blk, DIM), lambda i: (i, 0)),
            pl.BlockSpec((rows_blk, DIM), lambda i: (i, 0)),
            pl.BlockSpec((rows_blk, DIM), lambda i: (i, 0)),
            pl.BlockSpec((DIM, DIM), lambda i: (0, 0)),
            pl.BlockSpec((DIM, DIM), lambda i: (0, 0)),
            pl.BlockSpec((1, DIM), lambda i: (0, 0)),
            pl.BlockSpec((1, DIM), lambda i: (0, 0)),
        ],
        out_specs=pl.BlockSpec((rows_blk, DIM), lambda i: (i, 0)),
        out_shape=jax.ShapeDtypeStruct((NODES, DIM), jnp.float32),
    )(ent, parts[0, :NODES], parts[1, :NODES], W_self, W_nei,
      b_self.reshape(1, DIM), b_nei.reshape(1, DIM))

    return (out_ent, out_rel)


# speedup vs baseline: 1.1687x; 1.1587x over previous
import functools

import jax
import jax.numpy as jnp
from jax import lax
from jax.experimental import pallas as pl
from jax.experimental.pallas import tpu as pltpu
from jax.experimental.pallas import tpu_sc as plsc

DIM = 128
NODES = 10000
NREL = 500
NRELP = 512
EDGES = 320000

NC = 2
NS = 16
NW = NC * NS
CHUNK = 56
NCHUNK = 180
EPW = NCHUNK * CHUNK
EDGES_PAD = NW * EPW
NPAD = 10240
RPT = NPAD // NS
WB = 40
LANES = 16
NBUF = 3
IR = 4
PERIOD = 12


def _rel_body(rel_ref, wrel_ref, brel_ref, gate_ref, outrel_ref):
    r = rel_ref[...]
    gate_ref[...] = 1.0 / (1.0 + jnp.exp(-r))
    acc = lax.dot_general(r[:NREL], wrel_ref[...], (((1,), (1,)), ((), ())),
                          preferred_element_type=jnp.float32)
    outrel_ref[...] = acc + brel_ref[...]


def _ent_body(ent_ref, p0_ref, p1_ref, ws_ref, wn_ref, bs_ref, bn_ref,
              out_ref):
    agg = p0_ref[0] + p1_ref[0]
    acc = lax.dot_general(ent_ref[...], ws_ref[...], (((1,), (1,)), ((), ())),
                          preferred_element_type=jnp.float32)
    acc = acc + lax.dot_general(agg, wn_ref[...], (((1,), (1,)), ((), ())),
                                preferred_element_type=jnp.float32)
    out_ref[...] = jnp.maximum(acc + bs_ref[...] + bn_ref[...], 0.0)


def _sc_body(src_hbm, typ_hbm, dst_hbm, ent_hbm, gate_hbm, out_hbm,
             rows0, rows1, rows2, gat0, gat1, gat2,
             srcc0, srcc1, srcc2, srcc3, typc0, typc1, typc2, typc3,
             dstc0, dstc1, dstc2, dstc3,
             agg_sh,
             gsem0, gsem1, gsem2, ssem0, ssem1, ssem2,
             isem0, isem1, isem2, isem3):
    rows = [rows0, rows1, rows2]
    gat = [gat0, gat1, gat2]
    srcc = [srcc0, srcc1, srcc2, srcc3]
    typc = [typc0, typc1, typc2, typc3]
    dstc = [dstc0, dstc1, dstc2, dstc3]
    gsem = [gsem0, gsem1, gsem2]
    ssem = [ssem0, ssem1, ssem2]
    isem = [isem0, isem1, isem2, isem3]

    c = lax.axis_index("c")
    s = lax.axis_index("s")
    wid = s * NC + c
    ebase = pl.multiple_of(wid * EPW, CHUNK)

    def idx_descs(j, r):
        goff = pl.multiple_of(ebase + j * CHUNK, CHUNK)
        esl = pl.ds(goff, CHUNK)
        return (
            pltpu.make_async_copy(src_hbm.at[esl], srcc[r], isem[r]),
            pltpu.make_async_copy(typ_hbm.at[esl], typc[r], isem[r]),
            pltpu.make_async_copy(dst_hbm.at[esl], dstc[r], isem[r]),
        )

    def gather_descs(p, r):
        return (
            pltpu.make_async_copy(ent_hbm.at[srcc[r]], rows[p], gsem[p]),
            pltpu.make_async_copy(gate_hbm.at[typc[r]], gat[p], gsem[p]),
        )

    def scatter_desc(p, r):
        return pltpu.make_async_copy(rows[p], agg_sh.at[dstc[r]], ssem[p])

    zero = jnp.zeros((LANES,), jnp.float32)

    @pl.loop(0, WB)
    def _zero_rows(i):
        for k in range(DIM // LANES):
            rows0[i, pl.ds(k * LANES, LANES)] = zero

    base = s * RPT
    for b in range(RPT // WB):
        pltpu.sync_copy(rows0.at[pl.ds(0, WB)],
                        agg_sh.at[pl.ds(base + b * WB, WB)])
    plsc.subcore_barrier()

    for d in idx_descs(0, 0):
        d.start()
    for d in idx_descs(1, 1):
        d.start()
    for d in idx_descs(2, 2):
        d.start()
    for d in idx_descs(0, 0):
        d.wait()
    for d in gather_descs(0, 0):
        d.start()
    for d in idx_descs(1, 1):
        d.wait()
    for d in gather_descs(1, 1):
        d.start()

    @pl.loop(0, NCHUNK, step=PERIOD)
    def _outer(j0):
        for q in range(PERIOD):
            j = j0 + q
            p = q % NBUF
            pg = (q + 2) % NBUF
            r = q % IR
            rg = (q + 2) % IR
            rn = (q + 3) % IR

            for d in gather_descs(p, r):
                d.wait()

            @plsc.parallel_loop(0, CHUNK, unroll=4)
            def _row(i):
                for k in range(DIM // LANES):
                    sl = pl.ds(k * LANES, LANES)
                    rows[p][i, sl] = rows[p][i, sl] * gat[p][i, sl]

            @pl.when(j > 0)
            def _wait_prev_scatter():
                scatter_desc(pg, rn).wait()

            @pl.when(j + 2 < NCHUNK)
            def _issue_gather():
                for d in idx_descs(j + 2, rg):
                    d.wait()
                for d in gather_descs(pg, rg):
                    d.start()

            @pl.when(j + 3 < NCHUNK)
            def _issue_idx():
                for d in idx_descs(j + 3, rn):
                    d.start()

            pltpu.async_copy(rows[p], agg_sh.at[dstc[r]], ssem[p], add=True)

    scatter_desc((NCHUNK - 1) % NBUF, (NCHUNK - 1) % IR).wait()
    plsc.subcore_barrier()

    for b in range(RPT // WB):
        sl = pl.ds(base + b * WB, WB)
        pltpu.sync_copy(agg_sh.at[sl], rows0.at[pl.ds(0, WB)])
        pltpu.sync_copy(rows0.at[pl.ds(0, WB)], out_hbm.at[c, sl])


_sc_edges = functools.partial(
    pl.kernel,
    out_type=jax.ShapeDtypeStruct((NC, NPAD, DIM), jnp.float32),
    mesh=plsc.VectorSubcoreMesh(core_axis_name="c", subcore_axis_name="s"),
    scratch_types=(
        [pltpu.VMEM((CHUNK, DIM), jnp.float32) for _ in range(6)]
        + [pltpu.VMEM((CHUNK,), jnp.int32) for _ in range(12)]
        + [pltpu.VMEM_SHARED((NPAD, DIM), jnp.float32)]
        + [pltpu.SemaphoreType.DMA for _ in range(10)]
    ),
)(_sc_body)


@jax.jit
def kernel(ent, rel, edge_index, edge_type, W_self, b_self, W_nei, b_nei,
           W_rel, b_rel):
    npad = EDGES_PAD - EDGES
    src = jnp.concatenate(
        [edge_index[0].astype(jnp.int32), jnp.zeros((npad,), jnp.int32)])
    dst = jnp.concatenate(
        [edge_index[1].astype(jnp.int32), jnp.full((npad,), NODES, jnp.int32)])
    typ = jnp.concatenate(
        [edge_type.astype(jnp.int32), jnp.zeros((npad,), jnp.int32)])

    gate, out_rel = pl.pallas_call(
        _rel_body,
        out_shape=[
            jax.ShapeDtypeStruct((NRELP, DIM), jnp.float32),
            jax.ShapeDtypeStruct((NREL, DIM), jnp.float32),
        ],
    )(jnp.concatenate([rel, jnp.zeros((NRELP - NREL, DIM), jnp.float32)]),
      W_rel, b_rel.reshape(1, DIM))

    parts = _sc_edges(src, typ, dst, ent, gate)

    rows_blk = 2000
    grid = NODES // rows_blk
    out_ent = pl.pallas_call(
        _ent_body,
        grid=(grid,),
        in_specs=[
            pl.BlockSpec((rows_blk, DIM), lambda i: (i, 0)),
            pl.BlockSpec((1, rows_blk, DIM), lambda i: (0, i, 0)),
            pl.BlockSpec((1, rows_blk, DIM), lambda i: (1, i, 0)),
            pl.BlockSpec((DIM, DIM), lambda i: (0, 0)),
            pl.BlockSpec((DIM, DIM), lambda i: (0, 0)),
            pl.BlockSpec((1, DIM), lambda i: (0, 0)),
            pl.BlockSpec((1, DIM), lambda i: (0, 0)),
        ],
        out_specs=pl.BlockSpec((rows_blk, DIM), lambda i: (i, 0)),
        out_shape=jax.ShapeDtypeStruct((NODES, DIM), jnp.float32),
    )(ent, parts, parts, W_self, W_nei,
      b_self.reshape(1, DIM), b_nei.reshape(1, DIM))

    return (out_ent, out_rel)


# speedup vs baseline: 1.2085x; 1.0340x over previous
import functools

import jax
import jax.numpy as jnp
from jax import lax
from jax.experimental import pallas as pl
from jax.experimental.pallas import tpu as pltpu
from jax.experimental.pallas import tpu_sc as plsc

DIM = 128
NODES = 10000
NREL = 500
NRELP = 512
EDGES = 320000

NC = 2
NS = 16
NW = NC * NS
CHUNK = 56
NCHUNK = 180
EPW = NCHUNK * CHUNK
EDGES_PAD = NW * EPW
NPAD = 10240
RPT = NPAD // NS
WB = 40
LANES = 16
NBUF = 3
IR = 4
PERIOD = 12


def _rel_body(rel_ref, wrel_ref, brel_ref, gate_ref, outrel_ref):
    r = rel_ref[...]
    gate_ref[...] = 1.0 / (1.0 + jnp.exp(-r))
    acc = lax.dot_general(r, wrel_ref[...], (((1,), (1,)), ((), ())),
                          preferred_element_type=jnp.float32)
    outrel_ref[...] = acc + brel_ref[...]


def _ent_body(ent_ref, p0_ref, p1_ref, ws_ref, wn_ref, bs_ref, bn_ref,
              out_ref):
    agg = p0_ref[...] + p1_ref[...]
    acc = lax.dot_general(ent_ref[...], ws_ref[...], (((1,), (1,)), ((), ())),
                          preferred_element_type=jnp.float32)
    acc = acc + lax.dot_general(agg, wn_ref[...], (((1,), (1,)), ((), ())),
                                preferred_element_type=jnp.float32)
    out_ref[...] = jnp.maximum(acc + bs_ref[...] + bn_ref[...], 0.0)


def _sc_body(src_hbm, typ_hbm, dst_hbm, ent_hbm, gate_hbm, out_hbm,
             rows0, rows1, rows2, gat0, gat1, gat2,
             srcc0, srcc1, srcc2, srcc3, typc0, typc1, typc2, typc3,
             dstc0, dstc1, dstc2, dstc3,
             agg_sh,
             gsem0, gsem1, gsem2, ssem0, ssem1, ssem2,
             isem0, isem1, isem2, isem3):
    rows = [rows0, rows1, rows2]
    gat = [gat0, gat1, gat2]
    srcc = [srcc0, srcc1, srcc2, srcc3]
    typc = [typc0, typc1, typc2, typc3]
    dstc = [dstc0, dstc1, dstc2, dstc3]
    gsem = [gsem0, gsem1, gsem2]
    ssem = [ssem0, ssem1, ssem2]
    isem = [isem0, isem1, isem2, isem3]

    c = lax.axis_index("c")
    s = lax.axis_index("s")
    wid = s * NC + c
    ebase = pl.multiple_of(wid * EPW, CHUNK)

    def idx_descs(j, r):
        goff = pl.multiple_of(ebase + j * CHUNK, CHUNK)
        esl = pl.ds(goff, CHUNK)
        return (
            pltpu.make_async_copy(src_hbm.at[esl], srcc[r], isem[r]),
            pltpu.make_async_copy(typ_hbm.at[esl], typc[r], isem[r]),
            pltpu.make_async_copy(dst_hbm.at[esl], dstc[r], isem[r]),
        )

    def gather_descs(p, r):
        return (
            pltpu.make_async_copy(ent_hbm.at[srcc[r]], rows[p], gsem[p]),
            pltpu.make_async_copy(gate_hbm.at[typc[r]], gat[p], gsem[p]),
        )

    def scatter_desc(p, r):
        return pltpu.make_async_copy(rows[p], agg_sh.at[dstc[r]], ssem[p])

    zero = jnp.zeros((LANES,), jnp.float32)

    @pl.loop(0, WB)
    def _zero_rows(i):
        for k in range(DIM // LANES):
            rows0[i, pl.ds(k * LANES, LANES)] = zero

    base = s * RPT
    for b in range(RPT // WB):
        pltpu.sync_copy(rows0.at[pl.ds(0, WB)],
                        agg_sh.at[pl.ds(base + b * WB, WB)])
    plsc.subcore_barrier()

    for d in idx_descs(0, 0):
        d.start()
    for d in idx_descs(1, 1):
        d.start()
    for d in idx_descs(2, 2):
        d.start()
    for d in idx_descs(0, 0):
        d.wait()
    for d in gather_descs(0, 0):
        d.start()
    for d in idx_descs(1, 1):
        d.wait()
    for d in gather_descs(1, 1):
        d.start()

    @pl.loop(0, NCHUNK, step=PERIOD)
    def _outer(j0):
        for q in range(PERIOD):
            j = j0 + q
            p = q % NBUF
            pg = (q + 2) % NBUF
            r = q % IR
            rg = (q + 2) % IR
            rn = (q + 3) % IR

            for d in gather_descs(p, r):
                d.wait()

            @plsc.parallel_loop(0, CHUNK, unroll=4)
            def _row(i):
                for k in range(DIM // LANES):
                    sl = pl.ds(k * LANES, LANES)
                    rows[p][i, sl] = rows[p][i, sl] * gat[p][i, sl]

            @pl.when(j > 0)
            def _wait_prev_scatter():
                scatter_desc(pg, rn).wait()

            @pl.when(j + 2 < NCHUNK)
            def _issue_gather():
                for d in idx_descs(j + 2, rg):
                    d.wait()
                for d in gather_descs(pg, rg):
                    d.start()

            @pl.when(j + 3 < NCHUNK)
            def _issue_idx():
                for d in idx_descs(j + 3, rn):
                    d.start()

            pltpu.async_copy(rows[p], agg_sh.at[dstc[r]], ssem[p], add=True)

    scatter_desc((NCHUNK - 1) % NBUF, (NCHUNK - 1) % IR).wait()
    plsc.subcore_barrier()

    for b in range(RPT // WB):
        sl = pl.ds(base + b * WB, WB)
        pltpu.sync_copy(agg_sh.at[sl], rows0.at[pl.ds(0, WB)])
        pltpu.sync_copy(rows0.at[pl.ds(0, WB)], out_hbm.at[c, sl])


_sc_edges = functools.partial(
    pl.kernel,
    out_type=jax.ShapeDtypeStruct((NC, NPAD, DIM), jnp.float32),
    mesh=plsc.VectorSubcoreMesh(core_axis_name="c", subcore_axis_name="s"),
    scratch_types=(
        [pltpu.VMEM((CHUNK, DIM), jnp.float32) for _ in range(6)]
        + [pltpu.VMEM((CHUNK,), jnp.int32) for _ in range(12)]
        + [pltpu.VMEM_SHARED((NPAD, DIM), jnp.float32)]
        + [pltpu.SemaphoreType.DMA for _ in range(10)]
    ),
)(_sc_body)


@jax.jit
def kernel(ent, rel, edge_index, edge_type, W_self, b_self, W_nei, b_nei,
           W_rel, b_rel):
    npad = EDGES_PAD - EDGES
    src = jnp.concatenate(
        [edge_index[0].astype(jnp.int32), jnp.zeros((npad,), jnp.int32)])
    dst = jnp.concatenate(
        [edge_index[1].astype(jnp.int32), jnp.full((npad,), NODES, jnp.int32)])
    typ = jnp.concatenate(
        [edge_type.astype(jnp.int32), jnp.zeros((npad,), jnp.int32)])

    gate, out_rel = pl.pallas_call(
        _rel_body,
        out_shape=[
            jax.ShapeDtypeStruct((NREL, DIM), jnp.float32),
            jax.ShapeDtypeStruct((NREL, DIM), jnp.float32),
        ],
    )(rel, W_rel, b_rel.reshape(1, DIM))

    parts = _sc_edges(src, typ, dst, ent, gate)

    rows_blk = 2000
    grid = NODES // rows_blk
    out_ent = pl.pallas_call(
        _ent_body,
        grid=(grid,),
        in_specs=[
            pl.BlockSpec((rows_blk, DIM), lambda i: (i, 0)),
            pl.BlockSpec((rows_blk, DIM), lambda i: (i, 0)),
            pl.BlockSpec((rows_blk, DIM), lambda i: (i, 0)),
            pl.BlockSpec((DIM, DIM), lambda i: (0, 0)),
            pl.BlockSpec((DIM, DIM), lambda i: (0, 0)),
            pl.BlockSpec((1, DIM), lambda i: (0, 0)),
            pl.BlockSpec((1, DIM), lambda i: (0, 0)),
        ],
        out_specs=pl.BlockSpec((rows_blk, DIM), lambda i: (i, 0)),
        out_shape=jax.ShapeDtypeStruct((NODES, DIM), jnp.float32),
    )(ent, parts[0, :NODES], parts[1, :NODES], W_self, W_nei,
      b_self.reshape(1, DIM), b_nei.reshape(1, DIM))

    return (out_ent, out_rel)
